# Initial kernel scaffold; baseline (speedup 1.0000x reference)
import functools

import jax
import jax.numpy as jnp
from jax import lax
from jax.experimental import pallas as pl
from jax.experimental.pallas import tpu as pltpu
from jax.experimental.pallas import tpu_sc as plsc

N = 10000
E = 320000
D = 128
L = 16
NC = 2
NS = 16
CH = 128
EPT = 20096
NCH = EPT // CH
E_PAD = EPT * NS
NROW_BLK = 128
N_PAD = 10240
BLK_PER_SC = N_PAD // NROW_BLK // NS
ROWS_OUT = N // NS


def _spmm_body(x_hbm, src_hbm, dst_hbm, w_hbm, out_hbm,
               src_v, dst_v, w_v, rows_v, accum, sem):
    c = lax.axis_index("c")
    s = lax.axis_index("s")

    def zero_rows(i, carry):
        z = jnp.zeros((L,), jnp.float32)
        for j in range(D // L):
            rows_v[i, pl.ds(j * L, L)] = z
        return carry

    lax.fori_loop(0, CH, zero_rows, 0)

    def zero_accum(k, carry):
        blk = (s * BLK_PER_SC + k) * NROW_BLK
        pltpu.sync_copy(rows_v, accum.at[pl.ds(blk, NROW_BLK)])
        return carry

    lax.fori_loop(0, BLK_PER_SC, zero_accum, 0)
    plsc.subcore_barrier()

    base = s * EPT

    def chunk_body(ci, carry):
        off = base + ci * CH
        pltpu.sync_copy(src_hbm.at[c, pl.ds(off, CH)], src_v)
        pltpu.sync_copy(dst_hbm.at[c, pl.ds(off, CH)], dst_v)
        pltpu.sync_copy(w_hbm.at[c, pl.ds(off, CH)], w_v)
        pltpu.async_copy(x_hbm.at[src_v], rows_v, sem).wait()

        def grp_body(g, carry2):
            wv = w_v[pl.ds(g * L, L)]
            for e in range(L):
                we = wv[e]
                r = g * L + e
                for j in range(D // L):
                    rows_v[r, pl.ds(j * L, L)] = rows_v[r, pl.ds(j * L, L)] * we
            return carry2

        lax.fori_loop(0, CH // L, grp_body, 0)
        pltpu.sync_copy(rows_v, accum.at[dst_v], add=True)
        return carry

    lax.fori_loop(0, NCH, chunk_body, 0)
    plsc.subcore_barrier()

    pltpu.sync_copy(accum.at[pl.ds(s * ROWS_OUT, ROWS_OUT)],
                    out_hbm.at[c, pl.ds(s * ROWS_OUT, ROWS_OUT)])


@jax.jit
def _sc_spmm(x, src, dst, w):
    mesh = plsc.VectorSubcoreMesh(core_axis_name="c", subcore_axis_name="s")
    f = functools.partial(
        pl.kernel,
        out_type=jax.ShapeDtypeStruct((NC, N, D), jnp.float32),
        mesh=mesh,
        scratch_types=[
            pltpu.VMEM((CH,), jnp.int32),
            pltpu.VMEM((CH,), jnp.int32),
            pltpu.VMEM((CH,), jnp.float32),
            pltpu.VMEM((CH, D), jnp.float32),
            pltpu.VMEM_SHARED((N_PAD, D), jnp.float32),
            pltpu.SemaphoreType.DMA,
        ],
    )(_spmm_body)
    return f(x, src, dst, w)


def kernel(x, edge_index1, edge_weight1, edge_index2, edge_weight2):
    pad = E_PAD - E
    src = jnp.pad(jnp.stack([edge_index1[1], edge_index2[1]]), ((0, 0), (0, pad)))
    dst = jnp.pad(jnp.stack([edge_index1[0], edge_index2[0]]), ((0, 0), (0, pad)))
    w = jnp.pad(jnp.stack([edge_weight1, edge_weight2]), ((0, 0), (0, pad)))
    out = _sc_spmm(x, src, dst, w)
    return jnp.concatenate([out[0], out[1]], axis=1)


# speedup vs baseline: 4.3648x; 4.3648x over previous
import functools

import jax
import jax.numpy as jnp
from jax import lax
from jax.experimental import pallas as pl
from jax.experimental.pallas import tpu as pltpu
from jax.experimental.pallas import tpu_sc as plsc

N = 10000
E = 320000
D = 128
L = 16
NC = 2
NS = 16
CH = 128
EPT = 20096
NCH = EPT // CH
E_PAD = EPT * NS
NROW_BLK = 128
N_PAD = 10240
BLK_PER_SC = N_PAD // NROW_BLK // NS
ROWS_OUT = N_PAD // NS


def _spmm_body(x_hbm, src_hbm, dst_hbm, w_hbm, out_hbm,
               src_v, dst_v, w_v, rows_v, accum, sem):
    c = lax.axis_index("c")
    s = lax.axis_index("s")

    def zero_rows(i, carry):
        z = jnp.zeros((L,), jnp.float32)
        for j in range(D // L):
            rows_v[i, pl.ds(j * L, L)] = z
        return carry

    lax.fori_loop(0, CH, zero_rows, 0)

    def zero_accum(k, carry):
        blk = (s * BLK_PER_SC + k) * NROW_BLK
        pltpu.sync_copy(rows_v, accum.at[pl.ds(blk, NROW_BLK)])
        return carry

    lax.fori_loop(0, BLK_PER_SC, zero_accum, 0)
    plsc.subcore_barrier()

    base = s * EPT

    def chunk_body(ci, carry):
        off = c * E_PAD + base + ci * CH
        pltpu.sync_copy(src_hbm.at[pl.ds(off, CH)], src_v)
        pltpu.sync_copy(dst_hbm.at[pl.ds(off, CH)], dst_v)
        pltpu.sync_copy(w_hbm.at[pl.ds(off, CH)], w_v)
        pltpu.async_copy(x_hbm.at[src_v], rows_v, sem).wait()

        def grp_body(g, carry2):
            wv = w_v[pl.ds(g * L, L)]
            for e in range(L):
                we = wv[e]
                r = g * L + e
                for j in range(D // L):
                    rows_v[r, pl.ds(j * L, L)] = rows_v[r, pl.ds(j * L, L)] * we
            return carry2

        lax.fori_loop(0, CH // L, grp_body, 0)
        pltpu.sync_copy(rows_v, accum.at[dst_v], add=True)
        return carry

    lax.fori_loop(0, NCH, chunk_body, 0)
    plsc.subcore_barrier()

    pltpu.sync_copy(accum.at[pl.ds(s * ROWS_OUT, ROWS_OUT)],
                    out_hbm.at[c, pl.ds(s * ROWS_OUT, ROWS_OUT)])


@jax.jit
def _sc_spmm(x, src, dst, w):
    mesh = plsc.VectorSubcoreMesh(core_axis_name="c", subcore_axis_name="s")
    f = functools.partial(
        pl.kernel,
        out_type=jax.ShapeDtypeStruct((NC, N_PAD, D), jnp.float32),
        mesh=mesh,
        scratch_types=[
            pltpu.VMEM((CH,), jnp.int32),
            pltpu.VMEM((CH,), jnp.int32),
            pltpu.VMEM((CH,), jnp.float32),
            pltpu.VMEM((CH, D), jnp.float32),
            pltpu.VMEM_SHARED((N_PAD, D), jnp.float32),
            pltpu.SemaphoreType.DMA,
        ],
    )(_spmm_body)
    return f(x, src, dst, w)


def kernel(x, edge_index1, edge_weight1, edge_index2, edge_weight2):
    pad = E_PAD - E
    src = jnp.pad(jnp.stack([edge_index1[1], edge_index2[1]]),
                  ((0, 0), (0, pad))).reshape(-1)
    dst = jnp.pad(jnp.stack([edge_index1[0], edge_index2[0]]),
                  ((0, 0), (0, pad))).reshape(-1)
    w = jnp.pad(jnp.stack([edge_weight1, edge_weight2]),
                ((0, 0), (0, pad))).reshape(-1)
    out = _sc_spmm(x, src, dst, w)
    return jnp.concatenate([out[0, :N], out[1, :N]], axis=1)
